# baseline (speedup 1.0000x reference)
import functools

import jax
import jax.numpy as jnp
from jax import lax
from jax.experimental import pallas as pl
from jax.experimental.pallas import tpu as pltpu
from jax.experimental.pallas import tpu_sc as plsc

_B = 16384
_D = 32
_F = 128
_H1 = 256
_H2 = 128
_NC, _NS = 2, 16
_NW = _NC * _NS
_BPW = _B // _NW

_BLK = 2048


@functools.lru_cache(maxsize=None)
def _sc_gather_kernel():
    @functools.partial(
        pl.kernel,
        mesh=plsc.VectorSubcoreMesh(core_axis_name="c", subcore_axis_name="s"),
        compiler_params=pltpu.CompilerParams(use_tc_tiling_on_sc=False),
        out_type=[
            jax.ShapeDtypeStruct((_B, _D), jnp.float32),
            jax.ShapeDtypeStruct((_B, _D), jnp.float32),
        ],
        scratch_types=[
            pltpu.VMEM((_BPW,), jnp.int32),
            pltpu.VMEM((_BPW, _D), jnp.float32),
            pltpu.VMEM((_BPW,), jnp.int32),
            pltpu.VMEM((_BPW, _D), jnp.float32),
            pltpu.SemaphoreType.DMA,
            pltpu.SemaphoreType.DMA,
        ],
    )
    def _sc_gather(user_emb_hbm, uid_hbm, item_emb_hbm, iid_hbm, out_u, out_i,
                   uidx_v, urows_v, iidx_v, irows_v, usem, isem):
        wid = lax.axis_index("s") * _NC + lax.axis_index("c")
        base = wid * _BPW
        pltpu.sync_copy(uid_hbm.at[pl.ds(base, _BPW)], uidx_v)
        pltpu.sync_copy(iid_hbm.at[pl.ds(base, _BPW)], iidx_v)
        ucp = pltpu.async_copy(user_emb_hbm.at[uidx_v], urows_v, usem)
        icp = pltpu.async_copy(item_emb_hbm.at[iidx_v], irows_v, isem)
        ucp.wait()
        icp.wait()
        pltpu.sync_copy(urows_v, out_u.at[pl.ds(base, _BPW)])
        pltpu.sync_copy(irows_v, out_i.at[pl.ds(base, _BPW)])

    return _sc_gather


def _mlp_body(ue_ref, ie_ref, uf_ref, if_ref,
              W_uf_ref, b_uf_ref, W_if_ref, b_if_ref,
              W1_ref, b1_ref, W2_ref, b2_ref, w3t_ref, b3_ref, out_ref):
    ufe = jnp.dot(uf_ref[...], W_uf_ref[...],
                  preferred_element_type=jnp.float32) + b_uf_ref[...]
    ife = jnp.dot(if_ref[...], W_if_ref[...],
                  preferred_element_type=jnp.float32) + b_if_ref[...]
    x = jnp.concatenate([ue_ref[...], ufe, ie_ref[...], ife], axis=1)
    h1 = jnp.maximum(
        jnp.dot(x, W1_ref[...], preferred_element_type=jnp.float32)
        + b1_ref[...], 0.0)
    h2 = jnp.maximum(
        jnp.dot(h1, W2_ref[...], preferred_element_type=jnp.float32)
        + b2_ref[...], 0.0)
    out_ref[...] = (jnp.sum(h2 * w3t_ref[...], axis=1, keepdims=True)
                    + b3_ref[...])


def _tc_mlp(ue, ie, uf, if_, W_uf, b_uf, W_if, b_if, W1, b1, W2, b2, w3t, b3):
    grid = (_B // _BLK,)
    row_spec = lambda cols: pl.BlockSpec((_BLK, cols), lambda i: (i, 0))
    full_spec = lambda r, c: pl.BlockSpec((r, c), lambda i: (0, 0))
    return pl.pallas_call(
        _mlp_body,
        grid=grid,
        in_specs=[
            row_spec(_D), row_spec(_D), row_spec(_F), row_spec(_F),
            full_spec(_F, _D), full_spec(1, _D),
            full_spec(_F, _D), full_spec(1, _D),
            full_spec(4 * _D, _H1), full_spec(1, _H1),
            full_spec(_H1, _H2), full_spec(1, _H2),
            full_spec(1, _H2), full_spec(1, 1),
        ],
        out_specs=pl.BlockSpec((_BLK, 1), lambda i: (i, 0)),
        out_shape=jax.ShapeDtypeStruct((_B, 1), jnp.float32),
    )(ue, ie, uf, if_, W_uf, b_uf, W_if, b_if, W1, b1, W2, b2, w3t, b3)


def kernel(user_ids, item_ids, user_features, item_features, user_emb,
           item_emb, W_uf, b_uf, W_if, b_if, W1, b1, W2, b2, W3, b3):
    user_embeds, item_embeds = _sc_gather_kernel()(
        user_emb, user_ids.astype(jnp.int32), item_emb,
        item_ids.astype(jnp.int32))
    out = _tc_mlp(
        user_embeds, item_embeds, user_features, item_features,
        W_uf, b_uf.reshape(1, _D), W_if, b_if.reshape(1, _D),
        W1, b1.reshape(1, _H1), W2, b2.reshape(1, _H2),
        W3.reshape(1, _H2), b3.reshape(1, 1))
    return out.reshape(_B)
